# baseline (speedup 1.0000x reference)
import functools

import jax
import jax.numpy as jnp
from jax import lax
from jax.experimental import pallas as pl
from jax.experimental.pallas import tpu as pltpu
from jax.experimental.pallas import tpu_sc as plsc

EMBED_DIM = 64
NUM_CLS = 16
SEQ = 200
CW = 128
LANES = 16
NQ = EMBED_DIM // LANES


@functools.cache
def _sc_pool(batch):
  info = plsc.get_sparse_core_info()
  num_workers = info.num_cores * info.num_subcores
  bpw = batch // num_workers
  nchunk = bpw * SEQ // CW
  period = 3200 // CW
  rows_per_period = 3200 // SEQ
  nbuf = 5
  mesh = plsc.VectorSubcoreMesh(core_axis_name="c", subcore_axis_name="s")

  @functools.partial(
      pl.kernel,
      out_type=jax.ShapeDtypeStruct((batch, EMBED_DIM), jnp.float32),
      mesh=mesh,
      scratch_types=[
          pltpu.VMEM((nchunk, CW), jnp.int32),
          pltpu.VMEM((nbuf, CW, EMBED_DIM), jnp.float32),
          pltpu.VMEM((bpw, EMBED_DIM), jnp.float32),
          pltpu.SemaphoreType.DMA,
      ],
      compiler_params=pltpu.CompilerParams(use_tc_tiling_on_sc=False),
  )
  def sc_pool(ids_hbm, emb_hbm, out_hbm, idx_v, rows_v, pooled_v, sem):
    wid = lax.axis_index("s") * info.num_cores + lax.axis_index("c")
    pltpu.sync_copy(ids_hbm.at[pl.ds(wid * nchunk, nchunk)], idx_v)

    def dma(ck, buf):
      return pltpu.make_async_copy(
          emb_hbm.at[idx_v.at[ck]], rows_v.at[buf], sem
      )

    for c in range(nbuf - 1):
      dma(c, c).start()

    zeros = (jnp.zeros((LANES,), jnp.float32),) * NQ

    def reduce_span(buf, lo, hi, acc):
      def body(r, a):
        return tuple(
            a[q] + rows_v[buf, r, pl.ds(q * LANES, LANES)] for q in range(NQ)
        )

      return lax.fori_loop(lo, hi, body, acc, unroll=4)

    def store_row(row, acc):
      for q in range(NQ):
        pooled_v[row, pl.ds(q * LANES, LANES)] = acc[q] * (1.0 / SEQ)

    def outer(p, carry):
      chunk0 = p * period
      row0 = p * rows_per_period
      acc = zeros
      for c in range(period):
        buf = c % nbuf
        ck = chunk0 + c
        nxt = ck + nbuf - 1

        @pl.when(nxt < nchunk)
        def _():
          dma(nxt, (c + nbuf - 1) % nbuf).start()

        dma(ck, buf).wait()
        tok0 = CW * c
        m = tok0 // SEQ
        bnd = SEQ * (m + 1) - tok0
        if bnd <= CW:
          acc = reduce_span(buf, 0, bnd, acc)
          store_row(row0 + m, acc)
          acc = zeros
          if bnd < CW:
            acc = reduce_span(buf, bnd, CW, acc)
        else:
          acc = reduce_span(buf, 0, CW, acc)
      return carry

    lax.fori_loop(0, nchunk // period, outer, 0)
    pltpu.sync_copy(pooled_v, out_hbm.at[pl.ds(wid * bpw, bpw)])

  return sc_pool


def _tc_head(pooled, w_t, bias):
  def body(p_ref, w_ref, b_ref, o_ref):
    o_ref[...] = (
        jnp.dot(p_ref[...], w_ref[...], preferred_element_type=jnp.float32)
        + b_ref[...]
    )

  return pl.pallas_call(
      body,
      out_shape=jax.ShapeDtypeStruct((pooled.shape[0], NUM_CLS), jnp.float32),
  )(pooled, w_t, bias)


def kernel(text_ids, length, emb, W, b):
  del length
  ids = jnp.reshape(text_ids, (-1, 128))
  pooled = _sc_pool(text_ids.shape[0])(ids, emb)
  return _tc_head(pooled, W.T, b.reshape(1, NUM_CLS))
